# initial kernel scaffold (unmeasured)
import jax
import jax.numpy as jnp
from jax import lax
from jax.experimental import pallas as pl
from jax.experimental.pallas import tpu as pltpu


def kernel(
    x,
):
    def body(*refs):
        pass

    out_shape = jax.ShapeDtypeStruct(..., jnp.float32)
    return pl.pallas_call(body, out_shape=out_shape)(...)


# baseline (device time: 2211062 ns/iter reference)
import jax
import jax.numpy as jnp
from jax import lax
from jax.experimental import pallas as pl
from jax.experimental.pallas import tpu as pltpu

N_DEV = 4


def kernel(x):
    m_per, n = x.shape
    half = m_per // 2

    def body(x_ref, out_ref, copy_sem, send_cw, recv_cw, send_ccw, recv_ccw):
        my = lax.axis_index("i")
        left = (my + N_DEV - 1) % N_DEV
        right = (my + 1) % N_DEV

        barrier = pltpu.get_barrier_semaphore()
        for nbr in (left, right):
            pl.semaphore_signal(
                barrier, inc=1,
                device_id=(nbr,), device_id_type=pl.DeviceIdType.MESH,
            )
        pl.semaphore_wait(barrier, 2)

        local = pltpu.make_async_copy(
            x_ref, out_ref.at[pl.ds(my * m_per, m_per), :], copy_sem
        )
        local.start()

        def cw_slice(origin):
            return out_ref.at[pl.ds(origin * m_per, half), :]

        def ccw_slice(origin):
            return out_ref.at[pl.ds(origin * m_per + half, half), :]

        def send_cw_desc(h):
            origin = (my + N_DEV - h) % N_DEV
            src = x_ref.at[pl.ds(0, half), :] if h == 0 else cw_slice(origin)
            return pltpu.make_async_remote_copy(
                src_ref=src,
                dst_ref=cw_slice(origin),
                send_sem=send_cw.at[h],
                recv_sem=recv_cw.at[h],
                device_id=(right,),
                device_id_type=pl.DeviceIdType.MESH,
            )

        def send_ccw_desc(h):
            origin = (my + h) % N_DEV
            src = x_ref.at[pl.ds(half, half), :] if h == 0 else ccw_slice(origin)
            return pltpu.make_async_remote_copy(
                src_ref=src,
                dst_ref=ccw_slice(origin),
                send_sem=send_ccw.at[h],
                recv_sem=recv_ccw.at[h],
                device_id=(left,),
                device_id_type=pl.DeviceIdType.MESH,
            )

        def recv_cw_desc(h):
            origin = (my + N_DEV - 1 - h) % N_DEV
            return pltpu.make_async_remote_copy(
                src_ref=x_ref.at[pl.ds(0, half), :],
                dst_ref=cw_slice(origin),
                send_sem=send_cw.at[h],
                recv_sem=recv_cw.at[h],
                device_id=(right,),
                device_id_type=pl.DeviceIdType.MESH,
            )

        def recv_ccw_desc(h):
            origin = (my + 1 + h) % N_DEV
            return pltpu.make_async_remote_copy(
                src_ref=x_ref.at[pl.ds(half, half), :],
                dst_ref=ccw_slice(origin),
                send_sem=send_ccw.at[h],
                recv_sem=recv_ccw.at[h],
                device_id=(left,),
                device_id_type=pl.DeviceIdType.MESH,
            )

        sends = []
        s0, s1 = send_cw_desc(0), send_ccw_desc(0)
        s0.start()
        s1.start()
        sends += [s0, s1]
        for h in range(1, N_DEV - 1):
            recv_cw_desc(h - 1).wait_recv()
            recv_ccw_desc(h - 1).wait_recv()
            s0, s1 = send_cw_desc(h), send_ccw_desc(h)
            s0.start()
            s1.start()
            sends += [s0, s1]
        recv_cw_desc(N_DEV - 2).wait_recv()
        recv_ccw_desc(N_DEV - 2).wait_recv()

        for s in sends:
            s.wait_send()
        local.wait()

    return pl.pallas_call(
        body,
        out_shape=jax.ShapeDtypeStruct((N_DEV * m_per, n), x.dtype),
        in_specs=[pl.BlockSpec(memory_space=pl.ANY)],
        out_specs=pl.BlockSpec(memory_space=pl.ANY),
        scratch_shapes=[
            pltpu.SemaphoreType.DMA,
            pltpu.SemaphoreType.DMA((N_DEV - 1,)),
            pltpu.SemaphoreType.DMA((N_DEV - 1,)),
            pltpu.SemaphoreType.DMA((N_DEV - 1,)),
            pltpu.SemaphoreType.DMA((N_DEV - 1,)),
        ],
        compiler_params=pltpu.CompilerParams(collective_id=0),
    )(x)


# device time: 1258204 ns/iter; 1.7573x vs baseline; 1.7573x over previous
import jax
import jax.numpy as jnp
from jax import lax
from jax.experimental import pallas as pl
from jax.experimental.pallas import tpu as pltpu

N_DEV = 4


CHUNK = 2048


def kernel(x):
    m_per, n = x.shape
    half = m_per // 2

    def body(x_ref, out_ref, vbuf, in_sems, up_sems,
             send_cw, recv_cw, send_ccw, recv_ccw):
        my = lax.axis_index("i")
        left = (my + N_DEV - 1) % N_DEV
        right = (my + 1) % N_DEV

        barrier = pltpu.get_barrier_semaphore()
        for nbr in (left, right):
            pl.semaphore_signal(
                barrier, inc=1,
                device_id=(nbr,), device_id_type=pl.DeviceIdType.MESH,
            )
        pl.semaphore_wait(barrier, 2)

        def cw_slice(origin):
            return out_ref.at[pl.ds(origin * m_per, half), :]

        def ccw_slice(origin):
            return out_ref.at[pl.ds(origin * m_per + half, half), :]

        def send_cw_desc(h):
            origin = (my + N_DEV - h) % N_DEV
            src = x_ref.at[pl.ds(0, half), :] if h == 0 else cw_slice(origin)
            return pltpu.make_async_remote_copy(
                src_ref=src,
                dst_ref=cw_slice(origin),
                send_sem=send_cw.at[h],
                recv_sem=recv_cw.at[h],
                device_id=(right,),
                device_id_type=pl.DeviceIdType.MESH,
            )

        def send_ccw_desc(h):
            origin = (my + h) % N_DEV
            src = x_ref.at[pl.ds(half, half), :] if h == 0 else ccw_slice(origin)
            return pltpu.make_async_remote_copy(
                src_ref=src,
                dst_ref=ccw_slice(origin),
                send_sem=send_ccw.at[h],
                recv_sem=recv_ccw.at[h],
                device_id=(left,),
                device_id_type=pl.DeviceIdType.MESH,
            )

        def recv_cw_desc(h):
            origin = (my + N_DEV - 1 - h) % N_DEV
            return pltpu.make_async_remote_copy(
                src_ref=x_ref.at[pl.ds(0, half), :],
                dst_ref=cw_slice(origin),
                send_sem=send_cw.at[h],
                recv_sem=recv_cw.at[h],
                device_id=(right,),
                device_id_type=pl.DeviceIdType.MESH,
            )

        def recv_ccw_desc(h):
            origin = (my + 1 + h) % N_DEV
            return pltpu.make_async_remote_copy(
                src_ref=x_ref.at[pl.ds(half, half), :],
                dst_ref=ccw_slice(origin),
                send_sem=send_ccw.at[h],
                recv_sem=recv_ccw.at[h],
                device_id=(left,),
                device_id_type=pl.DeviceIdType.MESH,
            )

        sends = []
        s0, s1 = send_cw_desc(0), send_ccw_desc(0)
        s0.start()
        s1.start()
        sends += [s0, s1]

        base = my * m_per
        for k in range(m_per // CHUNK):
            down = pltpu.make_async_copy(
                x_ref.at[pl.ds(k * CHUNK, CHUNK), :],
                vbuf.at[k % 2], in_sems.at[k % 2])
            down.start()
            down.wait()
            up = pltpu.make_async_copy(
                vbuf.at[k % 2],
                out_ref.at[pl.ds(base + k * CHUNK, CHUNK), :],
                up_sems.at[k % 2])
            up.start()
            up.wait()

        for h in range(1, N_DEV - 1):
            recv_cw_desc(h - 1).wait_recv()
            recv_ccw_desc(h - 1).wait_recv()
            s0, s1 = send_cw_desc(h), send_ccw_desc(h)
            s0.start()
            s1.start()
            sends += [s0, s1]
        recv_cw_desc(N_DEV - 2).wait_recv()
        recv_ccw_desc(N_DEV - 2).wait_recv()

        for s in sends:
            s.wait_send()

    return pl.pallas_call(
        body,
        out_shape=jax.ShapeDtypeStruct((N_DEV * m_per, n), x.dtype),
        in_specs=[pl.BlockSpec(memory_space=pl.ANY)],
        out_specs=pl.BlockSpec(memory_space=pl.ANY),
        scratch_shapes=[
            pltpu.VMEM((2, CHUNK, n), x.dtype),
            pltpu.SemaphoreType.DMA((2,)),
            pltpu.SemaphoreType.DMA((2,)),
            pltpu.SemaphoreType.DMA((N_DEV - 1,)),
            pltpu.SemaphoreType.DMA((N_DEV - 1,)),
            pltpu.SemaphoreType.DMA((N_DEV - 1,)),
            pltpu.SemaphoreType.DMA((N_DEV - 1,)),
        ],
        compiler_params=pltpu.CompilerParams(collective_id=0),
    )(x)
